# Initial kernel scaffold; baseline (speedup 1.0000x reference)
import jax
import jax.numpy as jnp
from jax.experimental import pallas as pl


def kernel(query, key, value, Wq, bq, Wk, bk, Wv, bv, Wo, bo):
    raise NotImplementedError("write your pallas kernel here")


# speedup vs baseline: 1.7008x; 1.7008x over previous
import math

import jax
import jax.numpy as jnp
from jax.experimental import pallas as pl

B = 1
S = 2048
D_MODEL = 768
H = 12
DH = D_MODEL // H
BS = 256
NB = S // BS
TOPK = 3
SCALE = 1.0 / math.sqrt(DH)
NEG = -1e30

_DN = (((1,), (1,)), ((), ()))


def _qkv_kernel(xq_ref, xk_ref, xv_ref, wq_ref, bq_ref, wk_ref, bk_ref,
                wv_ref, bv_ref, q_ref, k_ref, v_ref, km_ref):
    q = jax.lax.dot_general(xq_ref[:], wq_ref[:], _DN,
                            preferred_element_type=jnp.float32) + bq_ref[:]
    k = jax.lax.dot_general(xk_ref[:], wk_ref[:], _DN,
                            preferred_element_type=jnp.float32) + bk_ref[:]
    v = jax.lax.dot_general(xv_ref[:], wv_ref[:], _DN,
                            preferred_element_type=jnp.float32) + bv_ref[:]
    q_ref[:] = q
    k_ref[:] = k
    v_ref[:] = v
    km_ref[0] = jnp.mean(k, axis=0, keepdims=True)


def _attn_kernel(q_ref, k_ref, v_ref, km_ref, o_ref):
    i = pl.program_id(1)
    qb = q_ref[0]

    gate = jax.lax.dot_general(qb, km_ref[0], _DN,
                               preferred_element_type=jnp.float32)
    jidx = jax.lax.broadcasted_iota(jnp.int32, (BS, NB), 1)
    gate = jnp.where(jidx < i, gate, NEG)
    rank = jnp.zeros((BS, NB), jnp.int32)
    for jp in range(NB):
        gp = gate[:, jp:jp + 1]
        ahead = (gp > gate) | ((gp == gate) & (jp < jidx))
        rank = rank + ahead.astype(jnp.int32)
    sel = ((rank < TOPK) & (jidx < i)).astype(jnp.float32)

    k_i = k_ref[0, pl.ds(i * BS, BS), :]
    v_i = v_ref[0, pl.ds(i * BS, BS), :]
    s = jax.lax.dot_general(qb, k_i, _DN,
                            preferred_element_type=jnp.float32) * SCALE
    rows = jax.lax.broadcasted_iota(jnp.int32, (BS, BS), 0)
    cols = jax.lax.broadcasted_iota(jnp.int32, (BS, BS), 1)
    s = jnp.where(cols <= rows, s, NEG)
    m_self = jnp.max(s, axis=1, keepdims=True)
    p = jnp.exp(s - m_self)
    l_self = jnp.sum(p, axis=1, keepdims=True)
    o_self = jnp.dot(p, v_i, preferred_element_type=jnp.float32) / l_self

    def body(j, carry):
        m, l, acc = carry
        k_j = k_ref[0, pl.ds(j * BS, BS), :]
        v_j = v_ref[0, pl.ds(j * BS, BS), :]
        sj = jax.lax.dot_general(qb, k_j, _DN,
                                 preferred_element_type=jnp.float32) * SCALE
        sel_j = jnp.sum(jnp.where(jidx == j, sel, 0.0), axis=1,
                        keepdims=True)
        sj = jnp.where(sel_j > 0.0, sj, NEG)
        m_new = jnp.maximum(m, jnp.max(sj, axis=1, keepdims=True))
        alpha = jnp.exp(m - m_new)
        pj = jnp.exp(sj - m_new) * sel_j
        l = l * alpha + jnp.sum(pj, axis=1, keepdims=True)
        acc = acc * alpha + jnp.dot(pj, v_j, preferred_element_type=jnp.float32)
        return m_new, l, acc

    m0 = jnp.full((BS, 1), NEG, jnp.float32)
    l0 = jnp.zeros((BS, 1), jnp.float32)
    a0 = jnp.zeros((BS, DH), jnp.float32)
    m, l, acc = jax.lax.fori_loop(0, i, body, (m0, l0, a0))
    o_moba = jnp.where(l > 0.0, acc / jnp.maximum(l, 1e-30), 0.0)

    o_ref[0] = o_self + o_moba


def _oproj_kernel(x_ref, w_ref, b_ref, o_ref):
    o_ref[:] = jax.lax.dot_general(
        x_ref[:], w_ref[:], _DN, preferred_element_type=jnp.float32) + b_ref[:]


def kernel(query, key, value, Wq, bq, Wk, bk, Wv, bv, Wo, bo):
    xq = query.reshape(S, D_MODEL)
    xk = key.reshape(S, D_MODEL)
    xv = value.reshape(S, D_MODEL)

    q, k, v, km = pl.pallas_call(
        _qkv_kernel,
        grid=(NB,),
        in_specs=[
            pl.BlockSpec((BS, D_MODEL), lambda m: (m, 0)),
            pl.BlockSpec((BS, D_MODEL), lambda m: (m, 0)),
            pl.BlockSpec((BS, D_MODEL), lambda m: (m, 0)),
            pl.BlockSpec((D_MODEL, D_MODEL), lambda m: (0, 0)),
            pl.BlockSpec((1, D_MODEL), lambda m: (0, 0)),
            pl.BlockSpec((D_MODEL, D_MODEL), lambda m: (0, 0)),
            pl.BlockSpec((1, D_MODEL), lambda m: (0, 0)),
            pl.BlockSpec((D_MODEL, D_MODEL), lambda m: (0, 0)),
            pl.BlockSpec((1, D_MODEL), lambda m: (0, 0)),
        ],
        out_specs=[
            pl.BlockSpec((BS, D_MODEL), lambda m: (m, 0)),
            pl.BlockSpec((BS, D_MODEL), lambda m: (m, 0)),
            pl.BlockSpec((BS, D_MODEL), lambda m: (m, 0)),
            pl.BlockSpec((1, 1, D_MODEL), lambda m: (m, 0, 0)),
        ],
        out_shape=[
            jax.ShapeDtypeStruct((S, D_MODEL), jnp.float32),
            jax.ShapeDtypeStruct((S, D_MODEL), jnp.float32),
            jax.ShapeDtypeStruct((S, D_MODEL), jnp.float32),
            jax.ShapeDtypeStruct((NB, 1, D_MODEL), jnp.float32),
        ],
    )(xq, xk, xv, Wq, bq.reshape(1, -1), Wk, bk.reshape(1, -1),
      Wv, bv.reshape(1, -1))

    qh = q.reshape(S, H, DH).transpose(1, 0, 2)
    kh = k.reshape(S, H, DH).transpose(1, 0, 2)
    vh = v.reshape(S, H, DH).transpose(1, 0, 2)
    kmh = km.reshape(NB, H, DH).transpose(1, 0, 2)

    attn = pl.pallas_call(
        _attn_kernel,
        grid=(H, NB),
        in_specs=[
            pl.BlockSpec((1, BS, DH), lambda h, i: (h, i, 0)),
            pl.BlockSpec((1, S, DH), lambda h, i: (h, 0, 0)),
            pl.BlockSpec((1, S, DH), lambda h, i: (h, 0, 0)),
            pl.BlockSpec((1, NB, DH), lambda h, i: (h, 0, 0)),
        ],
        out_specs=pl.BlockSpec((1, BS, DH), lambda h, i: (h, i, 0)),
        out_shape=jax.ShapeDtypeStruct((H, S, DH), jnp.float32),
    )(qh, kh, vh, kmh)

    attn2 = attn.transpose(1, 0, 2).reshape(S, D_MODEL)

    out = pl.pallas_call(
        _oproj_kernel,
        grid=(NB,),
        in_specs=[
            pl.BlockSpec((BS, D_MODEL), lambda m: (m, 0)),
            pl.BlockSpec((D_MODEL, D_MODEL), lambda m: (0, 0)),
            pl.BlockSpec((1, D_MODEL), lambda m: (0, 0)),
        ],
        out_specs=pl.BlockSpec((BS, D_MODEL), lambda m: (m, 0)),
        out_shape=jax.ShapeDtypeStruct((S, D_MODEL), jnp.float32),
    )(attn2, Wo, bo.reshape(1, -1))

    return out.reshape(B, S, D_MODEL)


# speedup vs baseline: 2.0608x; 1.2116x over previous
import math

import jax
import jax.numpy as jnp
from jax.experimental import pallas as pl
from jax.experimental.pallas import tpu as pltpu

B = 1
S = 2048
D_MODEL = 768
H = 12
DH = D_MODEL // H
BS = 256
NB = S // BS
TOPK = 3
SCALE = 1.0 / math.sqrt(DH)
NEG = -1e30

_DN = (((1,), (1,)), ((), ()))


def _moba_kernel(xq_ref, xk_ref, xv_ref, wq_ref, bq_ref, wk_ref, bk_ref,
                 wv_ref, bv_ref, wo_ref, bo_ref, o_ref,
                 k_sc, v_sc, km_sc):
    m = pl.program_id(0)

    q = jax.lax.dot_general(xq_ref[:], wq_ref[:], _DN,
                            preferred_element_type=jnp.float32) + bq_ref[:]
    k = jax.lax.dot_general(xk_ref[:], wk_ref[:], _DN,
                            preferred_element_type=jnp.float32) + bk_ref[:]
    v = jax.lax.dot_general(xv_ref[:], wv_ref[:], _DN,
                            preferred_element_type=jnp.float32) + bv_ref[:]
    k_sc[pl.ds(m * BS, BS), :] = k
    v_sc[pl.ds(m * BS, BS), :] = v
    km_sc[pl.ds(m, 1), :] = jnp.mean(k, axis=0, keepdims=True)

    jidx = jax.lax.broadcasted_iota(jnp.int32, (BS, NB), 1)
    rows = jax.lax.broadcasted_iota(jnp.int32, (BS, BS), 0)
    cols = jax.lax.broadcasted_iota(jnp.int32, (BS, BS), 1)

    outs = []
    for h in range(H):
        lo = h * DH
        qh = q[:, lo:lo + DH]
        km = km_sc[:, lo:lo + DH]

        gate = jax.lax.dot_general(qh, km, _DN,
                                   preferred_element_type=jnp.float32)
        gate = jnp.where(jidx < m, gate, NEG)
        rank = jnp.zeros((BS, NB), jnp.int32)
        for jp in range(NB):
            gp = gate[:, jp:jp + 1]
            ahead = (gp > gate) | ((gp == gate) & (jp < jidx))
            rank = rank + ahead.astype(jnp.int32)
        sel = ((rank < TOPK) & (jidx < m)).astype(jnp.float32)

        k_i = k_sc[pl.ds(m * BS, BS), lo:lo + DH]
        v_i = v_sc[pl.ds(m * BS, BS), lo:lo + DH]
        s = jax.lax.dot_general(qh, k_i, _DN,
                                preferred_element_type=jnp.float32) * SCALE
        s = jnp.where(cols <= rows, s, NEG)
        m_self = jnp.max(s, axis=1, keepdims=True)
        p = jnp.exp(s - m_self)
        l_self = jnp.sum(p, axis=1, keepdims=True)
        o_self = jnp.dot(p, v_i, preferred_element_type=jnp.float32) / l_self

        def body(j, carry, _lo=lo, _sel=sel, _qh=qh):
            mx, l, acc = carry
            k_j = k_sc[pl.ds(j * BS, BS), _lo:_lo + DH]
            v_j = v_sc[pl.ds(j * BS, BS), _lo:_lo + DH]
            sj = jax.lax.dot_general(_qh, k_j, _DN,
                                     preferred_element_type=jnp.float32) * SCALE
            sel_j = jnp.sum(jnp.where(jidx == j, _sel, 0.0), axis=1,
                            keepdims=True)
            sj = jnp.where(sel_j > 0.0, sj, NEG)
            m_new = jnp.maximum(mx, jnp.max(sj, axis=1, keepdims=True))
            alpha = jnp.exp(mx - m_new)
            pj = jnp.exp(sj - m_new) * sel_j
            l = l * alpha + jnp.sum(pj, axis=1, keepdims=True)
            acc = acc * alpha + jnp.dot(pj, v_j,
                                        preferred_element_type=jnp.float32)
            return m_new, l, acc

        m0 = jnp.full((BS, 1), NEG, jnp.float32)
        l0 = jnp.zeros((BS, 1), jnp.float32)
        a0 = jnp.zeros((BS, DH), jnp.float32)
        _, l, acc = jax.lax.fori_loop(0, m, body, (m0, l0, a0))
        o_moba = jnp.where(l > 0.0, acc / jnp.maximum(l, 1e-30), 0.0)

        outs.append(o_self + o_moba)

    combined = jnp.concatenate(outs, axis=1)
    o_ref[:] = jax.lax.dot_general(
        combined, wo_ref[:], _DN,
        preferred_element_type=jnp.float32) + bo_ref[:]


def kernel(query, key, value, Wq, bq, Wk, bk, Wv, bv, Wo, bo):
    xq = query.reshape(S, D_MODEL)
    xk = key.reshape(S, D_MODEL)
    xv = value.reshape(S, D_MODEL)

    row_spec = pl.BlockSpec((BS, D_MODEL), lambda mm: (mm, 0))
    w_spec = pl.BlockSpec((D_MODEL, D_MODEL), lambda mm: (0, 0))
    b_spec = pl.BlockSpec((1, D_MODEL), lambda mm: (0, 0))

    out = pl.pallas_call(
        _moba_kernel,
        grid=(NB,),
        in_specs=[row_spec, row_spec, row_spec,
                  w_spec, b_spec, w_spec, b_spec, w_spec, b_spec,
                  w_spec, b_spec],
        out_specs=row_spec,
        out_shape=jax.ShapeDtypeStruct((S, D_MODEL), jnp.float32),
        scratch_shapes=[
            pltpu.VMEM((S, D_MODEL), jnp.float32),
            pltpu.VMEM((S, D_MODEL), jnp.float32),
            pltpu.VMEM((NB, D_MODEL), jnp.float32),
        ],
    )(xq, xk, xv, Wq, bq.reshape(1, -1), Wk, bk.reshape(1, -1),
      Wv, bv.reshape(1, -1), Wo, bo.reshape(1, -1))

    return out.reshape(B, S, D_MODEL)
